# Initial kernel scaffold; baseline (speedup 1.0000x reference)
import jax
import jax.numpy as jnp
from jax.experimental import pallas as pl


def kernel(token_ids, weight):
    raise NotImplementedError("write your pallas kernel here")


# speedup vs baseline: 1.6841x; 1.6841x over previous
import functools

import jax
import jax.numpy as jnp
from jax import lax
from jax.experimental import pallas as pl
from jax.experimental.pallas import tpu as pltpu
from jax.experimental.pallas import tpu_sc as plsc

NUM_TOKENS = 16384 * 50
DIM = 64
NC, NS = 2, 16
NW = NC * NS
BPW = NUM_TOKENS // NW
CHUNK = 128
NCHUNK = BPW // CHUNK


def _emb_body(idx_hbm, table_hbm, out_hbm, idx_v, rows_v, sem):
    wid = lax.axis_index("s") * NC + lax.axis_index("c")
    base = wid * BPW
    pltpu.sync_copy(idx_hbm.at[wid], idx_v)

    def chunk_step(c, _):
        pltpu.async_copy(table_hbm.at[idx_v.at[c]], rows_v, sem).wait()
        pltpu.sync_copy(rows_v, out_hbm.at[pl.ds(base + c * CHUNK, CHUNK)])
        return _

    lax.fori_loop(0, NCHUNK, chunk_step, None)


@jax.jit
def _embedding_lookup(idx3, weight):
    mesh = plsc.VectorSubcoreMesh(core_axis_name="c", subcore_axis_name="s")
    k = functools.partial(
        pl.kernel,
        out_type=jax.ShapeDtypeStruct((NUM_TOKENS, DIM), jnp.float32),
        mesh=mesh,
        scratch_types=[
            pltpu.VMEM((NCHUNK, CHUNK), jnp.int32),
            pltpu.VMEM((CHUNK, DIM), jnp.float32),
            pltpu.SemaphoreType.DMA,
        ],
        compiler_params=pltpu.CompilerParams(use_tc_tiling_on_sc=False),
    )(_emb_body)
    return k(idx3, weight)


def kernel(token_ids, weight):
    idx3 = token_ids.astype(jnp.int32).reshape(NW, NCHUNK, CHUNK)
    out = _embedding_lookup(idx3, weight)
    return out.reshape(token_ids.shape + (DIM,))


# speedup vs baseline: 1.8773x; 1.1147x over previous
import functools

import jax
import jax.numpy as jnp
from jax import lax
from jax.experimental import pallas as pl
from jax.experimental.pallas import tpu as pltpu
from jax.experimental.pallas import tpu_sc as plsc

NUM_TOKENS = 16384 * 50
DIM = 64
NC, NS = 2, 16
NW = NC * NS
BPW = NUM_TOKENS // NW
CHUNK = 128
NCHUNK = BPW // CHUNK
NBUF = 4


def _emb_body(idx_hbm, table_hbm, out_hbm, idx_v,
              rows0, rows1, rows2, rows3,
              sg0, sg1, sg2, sg3, so0, so1, so2, so3):
    rows = (rows0, rows1, rows2, rows3)
    sg = (sg0, sg1, sg2, sg3)
    so = (so0, so1, so2, so3)
    wid = lax.axis_index("s") * NC + lax.axis_index("c")
    base = wid * BPW
    pltpu.sync_copy(idx_hbm.at[wid], idx_v)

    def gather(c, b):
        pltpu.async_copy(table_hbm.at[idx_v.at[c]], rows[b], sg[b])

    def write_out(c, b):
        pltpu.async_copy(rows[b], out_hbm.at[pl.ds(base + c * CHUNK, CHUNK)],
                         so[b])

    gather(0, 0)
    gather(1, 1)
    gather(2, 2)
    pltpu.make_async_copy(table_hbm.at[idx_v.at[0]], rows0, sg0).wait()
    write_out(0, 0)
    gather(3, 3)
    pltpu.make_async_copy(table_hbm.at[idx_v.at[1]], rows1, sg1).wait()
    write_out(1, 1)

    def group(g, _):
        for b in range(NBUF):
            i = NBUF * g + b
            j = i - 2
            bj = (b + 2) % NBUF
            pltpu.make_async_copy(
                rows[b], out_hbm.at[pl.ds(base + (i - NBUF) * CHUNK, CHUNK)],
                so[b]).wait()
            gather(i, b)
            pltpu.make_async_copy(table_hbm.at[idx_v.at[j]], rows[bj],
                                  sg[bj]).wait()
            write_out(j, bj)
        return _

    lax.fori_loop(1, NCHUNK // NBUF, group, None)

    for j in (NCHUNK - 2, NCHUNK - 1):
        bj = j % NBUF
        pltpu.make_async_copy(table_hbm.at[idx_v.at[j]], rows[bj],
                              sg[bj]).wait()
        write_out(j, bj)
    for j in range(NCHUNK - NBUF, NCHUNK):
        b = j % NBUF
        pltpu.make_async_copy(
            rows[b], out_hbm.at[pl.ds(base + j * CHUNK, CHUNK)], so[b]).wait()


@jax.jit
def _embedding_lookup(idx3, weight):
    mesh = plsc.VectorSubcoreMesh(core_axis_name="c", subcore_axis_name="s")
    k = functools.partial(
        pl.kernel,
        out_type=jax.ShapeDtypeStruct((NUM_TOKENS, DIM), jnp.float32),
        mesh=mesh,
        scratch_types=[
            pltpu.VMEM((NCHUNK, CHUNK), jnp.int32),
            pltpu.VMEM((CHUNK, DIM), jnp.float32),
            pltpu.VMEM((CHUNK, DIM), jnp.float32),
            pltpu.VMEM((CHUNK, DIM), jnp.float32),
            pltpu.VMEM((CHUNK, DIM), jnp.float32),
            pltpu.SemaphoreType.DMA,
            pltpu.SemaphoreType.DMA,
            pltpu.SemaphoreType.DMA,
            pltpu.SemaphoreType.DMA,
            pltpu.SemaphoreType.DMA,
            pltpu.SemaphoreType.DMA,
            pltpu.SemaphoreType.DMA,
            pltpu.SemaphoreType.DMA,
        ],
        compiler_params=pltpu.CompilerParams(use_tc_tiling_on_sc=False),
    )(_emb_body)
    return k(idx3, weight)


def kernel(token_ids, weight):
    idx3 = token_ids.astype(jnp.int32).reshape(NW, NCHUNK, CHUNK)
    out = _embedding_lookup(idx3, weight)
    return out.reshape(token_ids.shape + (DIM,))
